# baseline (speedup 1.0000x reference)
import functools

import numpy as np
import jax
from jax import lax
import jax.numpy as jnp
from jax.experimental import pallas as pl
from jax.experimental.pallas import tpu as pltpu
from jax.experimental.pallas import tpu_sc as plsc

_N, _S, _F = 4096, 128, 96
_BN = 128

_RUNS22 = ((6, 18), (21, 33), (36, 48), (51, 60), (63, 69), (75, 84), (87, 93))
_IDX2212 = ([0], [1, 2, 3], [4], [5, 6, 7], [8, 9], [10, 11], [12], [13],
            [14, 15, 16], [17], [18], [19, 20, 21])
_IDX127 = ([0, 1], [2, 3], [4, 5], [6, 7], [7, 8], [9, 10], [10, 11])
_IDX74 = ([0, 2], [1, 2], [3, 4], [5, 6])


def _tc_body(p_ref, x_ref, o32, o22):
    xs = x_ref[...] * p_ref[0]
    o32[...] = xs
    xt = jnp.transpose(xs, (1, 0, 2))
    o22[...] = jnp.concatenate([xt[a:b] for a, b in _RUNS22], axis=0)


def _tc_call(p, xt):
    return pl.pallas_call(
        _tc_body,
        grid=(_N // _BN,),
        in_specs=[
            pl.BlockSpec(memory_space=pltpu.SMEM),
            pl.BlockSpec((_BN, _F, _S), lambda i: (i, 0, 0)),
        ],
        out_specs=[
            pl.BlockSpec((_BN, _F, _S), lambda i: (i, 0, 0)),
            pl.BlockSpec((66, _BN, _S), lambda i: (0, i, 0)),
        ],
        out_shape=[
            jax.ShapeDtypeStruct((_N, 96, _S), jnp.float32),
            jax.ShapeDtypeStruct((66, _N, _S), jnp.float32),
        ],
    )(p, xt)


_NC, _NS = 2, 16
_NW = _NC * _NS
_C = 8
_CH = 4
_PER_W = _N // _NW
_CHUNKS = _PER_W // _C

_JU = (2, 3, 4, 5, 7, 8, 9, 10, 12, 13, 14, 15, 17, 18, 19, 21, 22, 25, 26,
       27, 29, 30)


def _sc_body(xt_hbm, p_hbm, o12, o7, o4, buf_in, b12, b7, b4, pbuf, sem):
    wid = lax.axis_index("s") * _NC + lax.axis_index("c")
    pltpu.sync_copy(p_hbm, pbuf)
    pv = pbuf[...]

    def half(n0, col):
        pltpu.sync_copy(xt_hbm.at[pl.ds(n0, _CH), :, :], buf_in)

        def lanes(kk, _):
            d = pl.ds(pl.multiple_of(kk * 16, 16), 16)
            for ci in range(_CH):
                for c in range(3):
                    v12 = []
                    for g, idx in enumerate(_IDX2212):
                        acc = buf_in[ci, 3 * _JU[idx[0]] + c, d]
                        for j in idx[1:]:
                            acc = acc + buf_in[ci, 3 * _JU[j] + c, d]
                        acc = acc * (pv * (1.0 / len(idx)))
                        v12.append(acc)
                        b12[3 * g + c, col + ci, d] = acc
                    v7 = []
                    for g, idx in enumerate(_IDX127):
                        acc = (v12[idx[0]] + v12[idx[1]]) * 0.5
                        v7.append(acc)
                        b7[3 * g + c, col + ci, d] = acc
                    for g, idx in enumerate(_IDX74):
                        b4[3 * g + c, col + ci, d] = \
                            (v7[idx[0]] + v7[idx[1]]) * 0.5
            return _

        lax.fori_loop(0, _S // 16, lanes, None)

    def chunk(it, _):
        n0 = wid * _PER_W + it * _C
        half(n0, 0)
        half(n0 + _CH, _CH)
        pltpu.sync_copy(b12, o12.at[:, pl.ds(n0, _C), :])
        pltpu.sync_copy(b7, o7.at[:, pl.ds(n0, _C), :])
        pltpu.sync_copy(b4, o4.at[:, pl.ds(n0, _C), :])
        return _

    lax.fori_loop(0, _CHUNKS, chunk, None)


def _sc_call(p, xt):
    f32 = jnp.float32
    mesh = plsc.VectorSubcoreMesh(core_axis_name="c", subcore_axis_name="s",
                                  num_cores=_NC, num_subcores=_NS)
    p16 = jnp.broadcast_to(p, (16,))
    k = pl.kernel(
        _sc_body,
        out_type=[
            jax.ShapeDtypeStruct((36, _N, _S), f32),
            jax.ShapeDtypeStruct((21, _N, _S), f32),
            jax.ShapeDtypeStruct((12, _N, _S), f32),
        ],
        mesh=mesh,
        scratch_types=[
            pltpu.VMEM((_CH, _F, _S), f32),
            pltpu.VMEM((36, _C, _S), f32),
            pltpu.VMEM((21, _C, _S), f32),
            pltpu.VMEM((12, _C, _S), f32),
            pltpu.VMEM((16,), f32),
            pltpu.SemaphoreType.DMA,
        ],
    )
    return k(xt, p16)


@jax.jit
def kernel(x, preproc):
    p = jnp.asarray(preproc, jnp.float32).reshape((1,))
    xt = jnp.transpose(x, (0, 2, 1))
    x32, x22 = _tc_call(p, xt)
    x12, x7, x4 = _sc_call(p, xt)
    tr = lambda o: jnp.transpose(o, (1, 0, 2))
    return (x32, tr(x22), tr(x12), tr(x7), tr(x4))


# speedup vs baseline: 1.6203x; 1.6203x over previous
import numpy as np
import jax
import jax.numpy as jnp
from jax.experimental import pallas as pl
from jax.experimental.pallas import tpu as pltpu

_N, _S, _F = 4096, 128, 96
_BN = 128

_RUNS22 = ((6, 18), (21, 33), (36, 48), (51, 60), (63, 69), (75, 84), (87, 93))
_IDX2212 = ([0], [1, 2, 3], [4], [5, 6, 7], [8, 9], [10, 11], [12], [13],
            [14, 15, 16], [17], [18], [19, 20, 21])
_IDX127 = ([0, 1], [2, 3], [4, 5], [6, 7], [7, 8], [9, 10], [10, 11])
_IDX74 = ([0, 2], [1, 2], [3, 4], [5, 6])


def _group_mean(x, groups):
    pieces = []
    for idx in groups:
        seg = x[3 * idx[0]:3 * idx[0] + 3]
        for j in idx[1:]:
            seg = seg + x[3 * j:3 * j + 3]
        if len(idx) > 1:
            seg = seg * (1.0 / len(idx))
        pieces.append(seg)
    return jnp.concatenate(pieces, axis=0)


def _body(p_ref, x_ref, o32, o22, o12, o7, o4):
    xs = x_ref[...] * p_ref[0]
    o32[...] = xs
    xt = jnp.transpose(xs, (1, 0, 2))
    x22 = jnp.concatenate([xt[a:b] for a, b in _RUNS22], axis=0)
    o22[...] = x22
    x12 = _group_mean(x22, _IDX2212)
    o12[...] = x12
    x7 = _group_mean(x12, _IDX127)
    o7[...] = x7
    x4 = _group_mean(x7, _IDX74)
    o4[...] = x4


def _dmajor_spec(d):
    return pl.BlockSpec((d, _BN, _S), lambda i: (0, i, 0))


@jax.jit
def kernel(x, preproc):
    p = jnp.asarray(preproc, jnp.float32).reshape((1,))
    xt = jnp.transpose(x, (0, 2, 1))
    f32 = jnp.float32
    out = pl.pallas_call(
        _body,
        grid=(_N // _BN,),
        in_specs=[
            pl.BlockSpec(memory_space=pltpu.SMEM),
            pl.BlockSpec((_BN, _F, _S), lambda i: (i, 0, 0)),
        ],
        out_specs=[
            pl.BlockSpec((_BN, _F, _S), lambda i: (i, 0, 0)),
            _dmajor_spec(66), _dmajor_spec(36), _dmajor_spec(21),
            _dmajor_spec(12),
        ],
        out_shape=[
            jax.ShapeDtypeStruct((_N, 96, _S), f32),
            jax.ShapeDtypeStruct((66, _N, _S), f32),
            jax.ShapeDtypeStruct((36, _N, _S), f32),
            jax.ShapeDtypeStruct((21, _N, _S), f32),
            jax.ShapeDtypeStruct((12, _N, _S), f32),
        ],
    )(p, xt)
    x32 = out[0]
    rest = tuple(jnp.transpose(o, (1, 0, 2)) for o in out[1:])
    return (x32,) + rest
